# baseline (speedup 1.0000x reference)
import functools

import jax
import jax.numpy as jnp
from jax import lax
from jax.experimental import pallas as pl
from jax.experimental.pallas import tpu as pltpu
from jax.experimental.pallas import tpu_sc as plsc

D_MODEL = 64
SCALE = 8.0

NC, NS = 2, 16
NW = NC * NS
ROWS = 4096 * 200
CHUNK = 128
CH_PER_W = ROWS // (NW * CHUNK)
NBUF = 4
LANES = 16


_mesh = plsc.VectorSubcoreMesh(
    core_axis_name="c", subcore_axis_name="s", num_cores=NC, num_subcores=NS
)


@functools.partial(
    pl.kernel,
    out_type=jax.ShapeDtypeStruct((ROWS, D_MODEL), jnp.float32),
    mesh=_mesh,
    scratch_types=(
        [pltpu.VMEM((CH_PER_W, CHUNK), jnp.int32)]
        + [pltpu.VMEM((CHUNK, D_MODEL), jnp.float32) for _ in range(NBUF)]
        + [pltpu.SemaphoreType.DMA for _ in range(2 * NBUF)]
    ),
    compiler_params=pltpu.CompilerParams(use_tc_tiling_on_sc=False),
)
def _embed_sc(tok_hbm, table_hbm, out_hbm, idx_v, *rest):
    bufs = rest[:NBUF]
    gsem = rest[NBUF : 2 * NBUF]
    osem = rest[2 * NBUF :]

    wid = lax.axis_index("s") * NC + lax.axis_index("c")
    cbase = wid * CH_PER_W
    rbase = cbase * CHUNK

    pltpu.sync_copy(tok_hbm.at[pl.ds(cbase, CH_PER_W)], idx_v)

    def gather_start(b, g):
        pltpu.async_copy(table_hbm.at[idx_v.at[g]], bufs[b], gsem[b])

    for b in range(NBUF):
        gather_start(b, b)

    def scale_buf(buf):
        def row(i, carry):
            for j in range(D_MODEL // LANES):
                sl = pl.ds(j * LANES, LANES)
                buf[i, sl] = buf[i, sl] * SCALE
            return carry

        lax.fori_loop(0, CHUNK, row, 0)

    def step(t, carry):
        g0 = t * NBUF
        for b in range(NBUF):
            g = g0 + b
            pltpu.make_async_copy(table_hbm.at[idx_v.at[g]], bufs[b], gsem[b]).wait()
            scale_buf(bufs[b])
            out_view = out_hbm.at[pl.ds(rbase + g * CHUNK, CHUNK)]
            pltpu.async_copy(bufs[b], out_view, osem[b])

            @pl.when(g + NBUF < CH_PER_W)
            def _refill(b=b, g=g, out_view=out_view):
                pltpu.make_async_copy(bufs[b], out_view, osem[b]).wait()
                gather_start(b, g + NBUF)

        return carry

    lax.fori_loop(0, CH_PER_W // NBUF, step, 0)

    for b in range(NBUF):
        g = CH_PER_W - NBUF + b
        out_view = out_hbm.at[pl.ds(rbase + g * CHUNK, CHUNK)]
        pltpu.make_async_copy(bufs[b], out_view, osem[b]).wait()


def kernel(tokens, table):
    n_seq, n_tok = tokens.shape
    tok2d = tokens.astype(jnp.int32).reshape(ROWS // CHUNK, CHUNK)
    out = _embed_sc(tok2d, table)
    return out.reshape(n_seq, n_tok, D_MODEL)
